# Initial kernel scaffold; baseline (speedup 1.0000x reference)
import jax
import jax.numpy as jnp
from jax.experimental import pallas as pl


def kernel(inputs, embedding):
    raise NotImplementedError("write your pallas kernel here")


# speedup vs baseline: 1.4199x; 1.4199x over previous
import functools

import jax
import jax.numpy as jnp
from jax import lax
from jax.experimental import pallas as pl
from jax.experimental.pallas import tpu as pltpu
from jax.experimental.pallas import tpu_sc as plsc

_K = 8192
_D = 256
_NT = 16384
_TM = 256
_NB = _NT // _TM

_NW = 32
_RPW = _NT // _NW
_CH = 128
_NCH = _RPW // _CH


def _tc_body(x_ref, e_ref, idx_ref, loss_ref, esq_ref):
    i = pl.program_id(0)

    @pl.when(i == 0)
    def _init():
        e = e_ref[...]
        esq_ref[...] = jnp.sum(e * e, axis=1).reshape(1, _K)
        loss_ref[0, 0] = 0.0

    x = x_ref[...]
    xsq = jnp.sum(x * x, axis=1, keepdims=True)
    m = lax.dot_general(x, e_ref[...], (((1,), (1,)), ((), ())),
                        preferred_element_type=jnp.float32)
    d = xsq - 2.0 * m + esq_ref[...]
    mins = jnp.min(d, axis=1, keepdims=True)
    cols = lax.broadcasted_iota(jnp.int32, (_TM, _K), 1)
    idx = jnp.min(jnp.where(d == mins, cols, _K), axis=1, keepdims=True)
    idx_ref[0] = idx
    loss_ref[0, 0] += jnp.sum(mins)


def _tc_argmin(x_flat, embedding):
    return pl.pallas_call(
        _tc_body,
        grid=(_NB,),
        in_specs=[
            pl.BlockSpec((_TM, _D), lambda i: (i, 0)),
            pl.BlockSpec((_K, _D), lambda i: (0, 0)),
        ],
        out_specs=[
            pl.BlockSpec((1, _TM, 1), lambda i: (i, 0, 0)),
            pl.BlockSpec((1, 1), lambda i: (0, 0), memory_space=pltpu.SMEM),
        ],
        out_shape=[
            jax.ShapeDtypeStruct((_NB, _TM, 1), jnp.int32),
            jax.ShapeDtypeStruct((1, 1), jnp.float32),
        ],
        scratch_shapes=[pltpu.VMEM((1, _K), jnp.float32)],
        compiler_params=pltpu.CompilerParams(
            dimension_semantics=("arbitrary",),
        ),
    )(x_flat, embedding)


def _sc_gather(embedding, idx2d):
    mesh = plsc.VectorSubcoreMesh(core_axis_name="c", subcore_axis_name="s")

    @functools.partial(
        pl.kernel,
        mesh=mesh,
        out_type=jax.ShapeDtypeStruct((_NT, _D), jnp.float32),
        scratch_types=[
            pltpu.VMEM((_NCH, _CH), jnp.int32),
            pltpu.VMEM((_CH, _D), jnp.float32),
            pltpu.VMEM((_CH, _D), jnp.float32),
            pltpu.SemaphoreType.DMA,
            pltpu.SemaphoreType.DMA,
        ],
    )
    def gather_kernel(table_hbm, idx_hbm, out_hbm, idx_v, buf0, buf1, sem0, sem1):
        wid = lax.axis_index("s") * 2 + lax.axis_index("c")
        pltpu.sync_copy(idx_hbm.at[pl.ds(wid * _NCH, _NCH)], idx_v)
        bufs = (buf0, buf1)
        sems = (sem0, sem1)
        cps = [None] * _NCH
        cps[0] = pltpu.async_copy(table_hbm.at[idx_v.at[0]], bufs[0], sems[0])
        for j in range(_NCH):
            if j + 1 < _NCH:
                cps[j + 1] = pltpu.async_copy(
                    table_hbm.at[idx_v.at[j + 1]], bufs[(j + 1) % 2],
                    sems[(j + 1) % 2])
            cps[j].wait()
            pltpu.sync_copy(bufs[j % 2],
                            out_hbm.at[pl.ds(wid * _RPW + j * _CH, _CH)])

    return gather_kernel(embedding, idx2d)


def kernel(inputs, embedding):
    b, c, t, h, w = inputs.shape
    x_flat = jnp.transpose(inputs, (0, 2, 3, 4, 1)).reshape(-1, c)
    idx3, loss_sum = _tc_argmin(x_flat, embedding)
    idx_flat = idx3.reshape(-1)
    q_flat = _sc_gather(embedding, idx_flat.reshape(_NT // _CH, _CH))
    quantized = jnp.transpose(q_flat.reshape(b, t, h, w, c), (0, 4, 1, 2, 3))
    codebook_loss = loss_sum[0, 0] / inputs.size
    commitment_loss = 0.25 * codebook_loss
    indices = idx_flat.reshape(b, t, h, w)
    return quantized, codebook_loss, commitment_loss, indices
